# baseline (speedup 1.0000x reference)
import functools

import jax
import jax.numpy as jnp
from jax import lax
from jax.experimental import pallas as pl
from jax.experimental.pallas import tpu as pltpu

_K = 8192
_D = 32
_N = 16384
_TB = 128
_COMMIT = 0.25


def _vq_body(x_ref, w_ref, x2_ref, w2_ref, enc_ref, q_ref, loss_ref, perp_ref,
             counts_ref, loss_acc_ref):
    i = pl.program_id(0)
    nsteps = pl.num_programs(0)

    x = x_ref[...]
    w = w_ref[...]

    ab = lax.dot_general(x, w, (((1,), (1,)), ((), ())),
                         preferred_element_type=jnp.float32)
    dist = x2_ref[...] + w2_ref[...] - 2.0 * ab

    half = _K // 2
    d0 = dist[:, :half]
    d1 = dist[:, half:]
    m0 = jnp.min(d0, axis=1)
    i0 = jnp.argmin(d0, axis=1)
    m1 = jnp.min(d1, axis=1)
    i1 = jnp.argmin(d1, axis=1) + half
    m0r = m0.astype(jnp.bfloat16).astype(jnp.float32)
    idx = jnp.where(m1 < m0r, i1, i0)
    cols = lax.broadcasted_iota(jnp.int32, (_TB, _K), 1)
    enc = jnp.where(cols == idx[:, None], 1.0, 0.0).astype(jnp.float32)
    enc_ref[...] = enc

    q = lax.dot_general(enc.astype(jnp.bfloat16), w.astype(jnp.bfloat16),
                        (((1,), (0,)), ((), ())),
                        preferred_element_type=jnp.float32)
    q_ref[...] = x + (q - x)

    @pl.when(i == 0)
    def _init():
        counts_ref[...] = jnp.zeros_like(counts_ref)
        loss_acc_ref[0] = 0.0

    counts_ref[...] += jnp.sum(enc, axis=0, keepdims=True)
    loss_acc_ref[0] += jnp.sum((q - x) ** 2)

    @pl.when(i == nsteps - 1)
    def _finish():
        loss_ref[0] = _COMMIT * loss_acc_ref[0] / (_N * _D)
        p = counts_ref[...] / _N
        perp_ref[0] = jnp.exp(-jnp.sum(p * jnp.log(p + 1e-10)))


@functools.partial(jax.jit, static_argnames=())
def _vq_fused(flat_x, w):
    grid = (_N // _TB,)
    enc, q, loss, perp = pl.pallas_call(
        _vq_body,
        grid=grid,
        in_specs=[
            pl.BlockSpec((_TB, _D), lambda i: (i, 0)),
            pl.BlockSpec((_K, _D), lambda i: (0, 0)),
            pl.BlockSpec((_TB, 1), lambda i: (i, 0)),
            pl.BlockSpec((1, _K), lambda i: (0, 0)),
        ],
        out_specs=[
            pl.BlockSpec((_TB, _K), lambda i: (i, 0)),
            pl.BlockSpec((_TB, _D), lambda i: (i, 0)),
            pl.BlockSpec(memory_space=pltpu.SMEM),
            pl.BlockSpec(memory_space=pltpu.SMEM),
        ],
        out_shape=[
            jax.ShapeDtypeStruct((_N, _K), jnp.float32),
            jax.ShapeDtypeStruct((_N, _D), jnp.float32),
            jax.ShapeDtypeStruct((1,), jnp.float32),
            jax.ShapeDtypeStruct((1,), jnp.float32),
        ],
        scratch_shapes=[
            pltpu.VMEM((1, _K), jnp.float32),
            pltpu.SMEM((1,), jnp.float32),
        ],
    )(flat_x, w, jnp.sum(flat_x ** 2, axis=1, keepdims=True),
      jnp.sum(w ** 2, axis=1)[None, :])
    return enc, q, loss, perp


def kernel(inputs, embedding_weight):
    input_shape = inputs.shape
    flat_x = inputs.reshape(-1, _D)
    enc, q, loss, perp = _vq_fused(flat_x, embedding_weight)
    return (loss.reshape(()), q.reshape(input_shape), perp.reshape(()), enc)


# speedup vs baseline: 1.1704x; 1.1704x over previous
import functools

import jax
import jax.numpy as jnp
from jax import lax
from jax.experimental import pallas as pl
from jax.experimental.pallas import tpu as pltpu

_K = 8192
_D = 32
_N = 16384
_TB = 256
_COMMIT = 0.25


def _vq_body(x_ref, wb_ref, x2_ref, w2_ref, enc_ref, q_ref, loss_ref, perp_ref,
             counts_ref, loss_acc_ref):
    i = pl.program_id(0)
    nsteps = pl.num_programs(0)

    x = x_ref[...]
    wb = wb_ref[...]

    ab = lax.dot_general(x.astype(jnp.bfloat16), wb, (((1,), (1,)), ((), ())),
                         preferred_element_type=jnp.float32)
    dist = x2_ref[...] + w2_ref[...] - 2.0 * ab

    half = _K // 2
    d0 = dist[:, :half]
    d1 = dist[:, half:]
    m0 = jnp.min(d0, axis=1)
    i0 = jnp.argmin(d0, axis=1)
    m1 = jnp.min(d1, axis=1)
    i1 = jnp.argmin(d1, axis=1) + half
    m0r = m0.astype(jnp.bfloat16).astype(jnp.float32)
    idx = jnp.where(m1 < m0r, i1, i0)
    cols = lax.broadcasted_iota(jnp.int32, (_TB, _K), 1)
    enc = jnp.where(cols == idx[:, None], 1.0, 0.0).astype(jnp.float32)
    enc_ref[...] = enc

    enc_b = enc.astype(jnp.bfloat16)
    q = lax.dot_general(enc_b, wb, (((1,), (0,)), ((), ())),
                        preferred_element_type=jnp.float32)
    q_ref[...] = x + (q - x)

    @pl.when(i == 0)
    def _init():
        counts_ref[...] = jnp.zeros_like(counts_ref)
        loss_acc_ref[0] = 0.0

    ones_row = jnp.ones((1, _TB), jnp.bfloat16)
    counts_ref[...] += lax.dot_general(ones_row, enc_b, (((1,), (0,)), ((), ())),
                                       preferred_element_type=jnp.float32)
    loss_acc_ref[0] += jnp.sum((q - x) ** 2)

    @pl.when(i == nsteps - 1)
    def _finish():
        loss_ref[0] = _COMMIT * loss_acc_ref[0] / (_N * _D)
        p = counts_ref[...] / _N
        perp_ref[0] = jnp.exp(-jnp.sum(p * jnp.log(p + 1e-10)))


@functools.partial(jax.jit, static_argnames=())
def _vq_fused(flat_x, w):
    grid = (_N // _TB,)
    enc, q, loss, perp = pl.pallas_call(
        _vq_body,
        grid=grid,
        in_specs=[
            pl.BlockSpec((_TB, _D), lambda i: (i, 0)),
            pl.BlockSpec((_K, _D), lambda i: (0, 0)),
            pl.BlockSpec((_TB, 1), lambda i: (i, 0)),
            pl.BlockSpec((1, _K), lambda i: (0, 0)),
        ],
        out_specs=[
            pl.BlockSpec((_TB, _K), lambda i: (i, 0)),
            pl.BlockSpec((_TB, _D), lambda i: (i, 0)),
            pl.BlockSpec(memory_space=pltpu.SMEM),
            pl.BlockSpec(memory_space=pltpu.SMEM),
        ],
        out_shape=[
            jax.ShapeDtypeStruct((_N, _K), jnp.float32),
            jax.ShapeDtypeStruct((_N, _D), jnp.float32),
            jax.ShapeDtypeStruct((1,), jnp.float32),
            jax.ShapeDtypeStruct((1,), jnp.float32),
        ],
        scratch_shapes=[
            pltpu.VMEM((1, _K), jnp.float32),
            pltpu.SMEM((1,), jnp.float32),
        ],
    )(flat_x, w.astype(jnp.bfloat16), jnp.sum(flat_x ** 2, axis=1, keepdims=True),
      jnp.sum(w ** 2, axis=1)[None, :])
    return enc, q, loss, perp


def kernel(inputs, embedding_weight):
    input_shape = inputs.shape
    flat_x = inputs.reshape(-1, _D)
    enc, q, loss, perp = _vq_fused(flat_x, embedding_weight)
    return (loss.reshape(()), q.reshape(input_shape), perp.reshape(()), enc)
